# Initial kernel scaffold; baseline (speedup 1.0000x reference)
import jax
import jax.numpy as jnp
from jax.experimental import pallas as pl


def kernel(x, weight):
    raise NotImplementedError("write your pallas kernel here")


# speedup vs baseline: 2.0830x; 2.0830x over previous
import jax
import jax.numpy as jnp
from jax.experimental import pallas as pl
from jax.experimental.pallas import tpu as pltpu


def _mm_kernel(x_ref, w_ref, o_ref):
    o_ref[...] = jnp.dot(
        x_ref[...], w_ref[...], preferred_element_type=jnp.float32
    ).astype(o_ref.dtype)


@jax.jit
def kernel(x, weight):
    *lead, K = x.shape
    N = weight.shape[0]
    x2d = x.reshape(-1, K)
    M = x2d.shape[0]

    xb = x2d.astype(jnp.bfloat16)
    wb = weight.T.astype(jnp.bfloat16)

    tm, tn = 1024, 1024
    out2d = pl.pallas_call(
        _mm_kernel,
        out_shape=jax.ShapeDtypeStruct((M, N), x.dtype),
        grid=(M // tm, N // tn),
        in_specs=[
            pl.BlockSpec((tm, K), lambda i, j: (i, 0)),
            pl.BlockSpec((K, tn), lambda i, j: (0, j)),
        ],
        out_specs=pl.BlockSpec((tm, tn), lambda i, j: (i, j)),
        compiler_params=pltpu.CompilerParams(
            dimension_semantics=("parallel", "parallel"),
            vmem_limit_bytes=48 << 20,
        ),
    )(xb, wb)
    return out2d.reshape(*lead, N)


# speedup vs baseline: 2.2886x; 1.0987x over previous
import jax
import jax.numpy as jnp
from jax.experimental import pallas as pl
from jax.experimental.pallas import tpu as pltpu


def _mm_kernel(x_ref, w_ref, o_ref):
    o_ref[...] = jnp.dot(
        x_ref[...].astype(jnp.bfloat16), w_ref[...],
        preferred_element_type=jnp.float32,
    )


@jax.jit
def kernel(x, weight):
    *lead, K = x.shape
    N = weight.shape[0]
    x2d = x.reshape(-1, K)
    M = x2d.shape[0]

    wb = weight.T.astype(jnp.bfloat16)

    tm, tn = 256, N // 2
    out2d = pl.pallas_call(
        _mm_kernel,
        out_shape=jax.ShapeDtypeStruct((M, N), x.dtype),
        grid=(N // tn, M // tm),
        in_specs=[
            pl.BlockSpec((tm, K), lambda j, i: (i, 0)),
            pl.BlockSpec((K, tn), lambda j, i: (0, j)),
        ],
        out_specs=pl.BlockSpec((tm, tn), lambda j, i: (i, j)),
        compiler_params=pltpu.CompilerParams(
            dimension_semantics=("parallel", "arbitrary"),
            vmem_limit_bytes=50 << 20,
        ),
    )(x2d, wb)
    return out2d.reshape(*lead, N)


# speedup vs baseline: 2.4876x; 1.0870x over previous
import jax
import jax.numpy as jnp
from jax.experimental import pallas as pl
from jax.experimental.pallas import tpu as pltpu


def _mm_kernel(x_ref, w_ref, o_ref):
    o_ref[...] = jax.lax.dot_general(
        x_ref[...].astype(jnp.bfloat16), w_ref[...],
        dimension_numbers=(((1,), (1,)), ((), ())),
        preferred_element_type=jnp.float32,
    )


@jax.jit
def kernel(x, weight):
    *lead, K = x.shape
    N = weight.shape[0]
    x2d = x.reshape(-1, K)
    M = x2d.shape[0]

    wb = weight.astype(jnp.bfloat16)

    tm, tn = 256, N // 2
    out2d = pl.pallas_call(
        _mm_kernel,
        out_shape=jax.ShapeDtypeStruct((M, N), x.dtype),
        grid=(N // tn, M // tm),
        in_specs=[
            pl.BlockSpec((tm, K), lambda j, i: (i, 0)),
            pl.BlockSpec((tn, K), lambda j, i: (j, 0)),
        ],
        out_specs=pl.BlockSpec((tm, tn), lambda j, i: (i, j)),
        compiler_params=pltpu.CompilerParams(
            dimension_semantics=("parallel", "arbitrary"),
            vmem_limit_bytes=50 << 20,
        ),
    )(x2d, wb)
    return out2d.reshape(*lead, N)


# speedup vs baseline: 2.6949x; 1.0833x over previous
import jax
import jax.numpy as jnp
from jax.experimental import pallas as pl
from jax.experimental.pallas import tpu as pltpu

_TM_HALF = 2048
_CHUNK = 256
_NCHUNK = _TM_HALF // _CHUNK
_TN = 512


def _mm_kernel(x_hbm, w_ref, o_ref, xb_ref, stage_ref, sems):
    i = pl.program_id(0)
    j = pl.program_id(1)

    @pl.when(j == 0)
    def _load_x():
        base = i * _TM_HALF

        def _copy(c, slot):
            return pltpu.make_async_copy(
                x_hbm.at[pl.ds(base + c * _CHUNK, _CHUNK), :],
                stage_ref.at[slot],
                sems.at[slot],
            )

        _copy(0, 0).start()

        def _body(c, carry):
            slot = jax.lax.rem(c, 2)

            @pl.when(c + 1 < _NCHUNK)
            def _():
                _copy(c + 1, 1 - slot).start()

            _copy(c, slot).wait()
            xb_ref[pl.ds(c * _CHUNK, _CHUNK), :] = (
                stage_ref[slot].astype(jnp.bfloat16))
            return carry

        jax.lax.fori_loop(0, _NCHUNK, _body, 0)

    o_ref[...] = jax.lax.dot_general(
        xb_ref[...], w_ref[...].astype(jnp.bfloat16),
        dimension_numbers=(((1,), (1,)), ((), ())),
        preferred_element_type=jnp.float32,
    )


@jax.jit
def kernel(x, weight):
    *lead, K = x.shape
    N = weight.shape[0]
    x2d = x.reshape(-1, K)
    M = x2d.shape[0]

    out2d = pl.pallas_call(
        _mm_kernel,
        out_shape=jax.ShapeDtypeStruct((M, N), x.dtype),
        grid=(M // _TM_HALF, N // _TN),
        in_specs=[
            pl.BlockSpec(memory_space=pl.ANY),
            pl.BlockSpec((_TN, K), lambda i, j: (j, 0)),
        ],
        out_specs=pl.BlockSpec((_TM_HALF, _TN), lambda i, j: (i, j)),
        scratch_shapes=[
            pltpu.VMEM((_TM_HALF, K), jnp.bfloat16),
            pltpu.VMEM((2, _CHUNK, K), jnp.float32),
            pltpu.SemaphoreType.DMA((2,)),
        ],
        compiler_params=pltpu.CompilerParams(
            dimension_semantics=("parallel", "arbitrary"),
            vmem_limit_bytes=54 << 20,
        ),
    )(x2d, weight)
    return out2d.reshape(*lead, N)
